# Initial kernel scaffold; baseline (speedup 1.0000x reference)
import functools

import jax
import jax.numpy as jnp
from jax import lax
from jax.experimental import pallas as pl
from jax.experimental.pallas import tpu as pltpu
from jax.experimental.pallas import tpu_sc as plsc

B = 16384
L = 200
DIM = 32
N = B * L

NC = 2
NS = 16
NW = NC * NS
PER_W = N // NW

CHUNK = 2048
NCHUNK = PER_W // CHUNK

_mesh = plsc.VectorSubcoreMesh(core_axis_name="c", subcore_axis_name="s")


@functools.partial(
    pl.kernel,
    mesh=_mesh,
    out_type=jax.ShapeDtypeStruct((N, DIM), jnp.float32),
    scratch_types=[
        pltpu.VMEM((CHUNK,), jnp.int32),
        pltpu.VMEM((CHUNK, DIM), jnp.float32),
        pltpu.SemaphoreType.DMA,
    ],
)
def _gather_kernel(idx_hbm, table_hbm, out_hbm, idx_v, rows_v, sem):
    wid = lax.axis_index("s") * NC + lax.axis_index("c")
    base = wid * PER_W

    def body(g, carry):
        off = base + g * CHUNK
        pltpu.sync_copy(idx_hbm.at[pl.ds(off, CHUNK)], idx_v)
        pltpu.async_copy(table_hbm.at[idx_v], rows_v, sem).wait()
        pltpu.sync_copy(rows_v, out_hbm.at[pl.ds(off, CHUNK)])
        return carry

    lax.fori_loop(0, NCHUNK, body, 0)


def kernel(x, table):
    flat = x.reshape(N).astype(jnp.int32)
    out = _gather_kernel(flat, table)
    return out.reshape(B, L, DIM)


# speedup vs baseline: 4.9463x; 4.9463x over previous
import functools

import jax
import jax.numpy as jnp
from jax import lax
from jax.experimental import pallas as pl
from jax.experimental.pallas import tpu as pltpu
from jax.experimental.pallas import tpu_sc as plsc

B = 16384
L = 200
DIM = 32
N = B * L

NC = 2
NS = 16
NW = NC * NS
PER_W = N // NW

CHUNK = 2048
NCHUNK = PER_W // CHUNK

_mesh = plsc.VectorSubcoreMesh(core_axis_name="c", subcore_axis_name="s")


@functools.partial(
    pl.kernel,
    mesh=_mesh,
    out_type=jax.ShapeDtypeStruct((N, DIM), jnp.float32),
    scratch_types=[
        pltpu.VMEM((CHUNK,), jnp.int32),
        pltpu.VMEM((CHUNK, DIM), jnp.float32),
        pltpu.SemaphoreType.DMA,
    ],
    compiler_params=pltpu.CompilerParams(use_tc_tiling_on_sc=False),
)
def _gather_kernel(idx_hbm, table_hbm, out_hbm, idx_v, rows_v, sem):
    wid = lax.axis_index("s") * NC + lax.axis_index("c")
    base = wid * PER_W

    def body(g, carry):
        off = base + g * CHUNK
        pltpu.sync_copy(idx_hbm.at[pl.ds(off, CHUNK)], idx_v)
        pltpu.async_copy(table_hbm.at[idx_v], rows_v, sem).wait()
        pltpu.sync_copy(rows_v, out_hbm.at[pl.ds(off, CHUNK)])
        return carry

    lax.fori_loop(0, NCHUNK, body, 0)


def kernel(x, table):
    flat = x.reshape(N).astype(jnp.int32)
    out = _gather_kernel(flat, table)
    return out.reshape(B, L, DIM)


# speedup vs baseline: 5.0370x; 1.0183x over previous
import functools

import jax
import jax.numpy as jnp
from jax import lax
from jax.experimental import pallas as pl
from jax.experimental.pallas import tpu as pltpu
from jax.experimental.pallas import tpu_sc as plsc

B = 16384
L = 200
DIM = 32
N = B * L

NC = 2
NS = 16
NW = NC * NS
PER_W = N // NW

CHUNK = 1600
NBUF = 2
NCHUNK = PER_W // CHUNK
NOUTER = NCHUNK // NBUF

_mesh = plsc.VectorSubcoreMesh(core_axis_name="c", subcore_axis_name="s")


@functools.partial(
    pl.kernel,
    mesh=_mesh,
    out_type=jax.ShapeDtypeStruct((N, DIM), jnp.float32),
    scratch_types=[
        pltpu.VMEM((CHUNK,), jnp.int32),
        pltpu.VMEM((CHUNK,), jnp.int32),
        pltpu.VMEM((CHUNK, DIM), jnp.float32),
        pltpu.VMEM((CHUNK, DIM), jnp.float32),
        pltpu.SemaphoreType.DMA,
        pltpu.SemaphoreType.DMA,
        pltpu.SemaphoreType.DMA,
        pltpu.SemaphoreType.DMA,
        pltpu.SemaphoreType.DMA,
    ],
    compiler_params=pltpu.CompilerParams(use_tc_tiling_on_sc=False),
)
def _gather_kernel(idx_hbm, table_hbm, out_hbm, idx_v0, idx_v1,
                   rows_v0, rows_v1, sem_i0, sem_i1, sem_g, sem_o0, sem_o1):
    wid = lax.axis_index("s") * NC + lax.axis_index("c")
    base = wid * PER_W
    idx_v = [idx_v0, idx_v1]
    rows_v = [rows_v0, rows_v1]
    sems_i = [sem_i0, sem_i1]
    sems_o = [sem_o0, sem_o1]

    for b in range(NBUF):
        pltpu.async_copy(
            idx_hbm.at[pl.ds(base + b * CHUNK, CHUNK)], idx_v[b], sems_i[b])

    def outer(go, carry):
        for b in range(NBUF):
            g = go * NBUF + b
            off = base + g * CHUNK
            pltpu.make_async_copy(
                idx_hbm.at[pl.ds(off, CHUNK)], idx_v[b], sems_i[b]).wait()
            @pl.when(go > 0)
            def _():
                pltpu.make_async_copy(
                    rows_v[b], out_hbm.at[pl.ds(off, CHUNK)],
                    sems_o[b]).wait()
            pltpu.async_copy(table_hbm.at[idx_v[b]], rows_v[b],
                             sem_g).wait()
            pltpu.async_copy(rows_v[b], out_hbm.at[pl.ds(off, CHUNK)],
                             sems_o[b])
            @pl.when(go < NOUTER - 1)
            def _():
                pltpu.async_copy(
                    idx_hbm.at[pl.ds(off + NBUF * CHUNK, CHUNK)],
                    idx_v[b], sems_i[b])
        return carry

    lax.fori_loop(0, NOUTER, outer, 0)

    for b in range(NBUF):
        off = base + (NCHUNK - NBUF + b) * CHUNK
        pltpu.make_async_copy(
            rows_v[b], out_hbm.at[pl.ds(off, CHUNK)], sems_o[b]).wait()


def kernel(x, table):
    flat = x.reshape(N).astype(jnp.int32)
    out = _gather_kernel(flat, table)
    return out.reshape(B, L, DIM)


# speedup vs baseline: 5.0504x; 1.0027x over previous
import functools

import jax
import jax.numpy as jnp
from jax import lax
from jax.experimental import pallas as pl
from jax.experimental.pallas import tpu as pltpu
from jax.experimental.pallas import tpu_sc as plsc

B = 16384
L = 200
DIM = 32
N = B * L

NC = 2
NS = 16
NW = NC * NS
PER_W = N // NW

CHUNK = 800
NBUF = 4
NCHUNK = PER_W // CHUNK
NOUTER = NCHUNK // NBUF

_mesh = plsc.VectorSubcoreMesh(core_axis_name="c", subcore_axis_name="s")


@functools.partial(
    pl.kernel,
    mesh=_mesh,
    out_type=jax.ShapeDtypeStruct((N, DIM), jnp.float32),
    scratch_types=(
        [pltpu.VMEM((CHUNK,), jnp.int32) for _ in range(NBUF)]
        + [pltpu.VMEM((CHUNK, DIM), jnp.float32) for _ in range(NBUF)]
        + [pltpu.SemaphoreType.DMA for _ in range(3 * NBUF)]
    ),
    compiler_params=pltpu.CompilerParams(use_tc_tiling_on_sc=False),
)
def _gather_kernel(idx_hbm, table_hbm, out_hbm, *scratch):
    idx_v = scratch[:NBUF]
    rows_v = scratch[NBUF:2 * NBUF]
    sems_i = scratch[2 * NBUF:3 * NBUF]
    sems_g = scratch[3 * NBUF:4 * NBUF]
    sems_o = scratch[4 * NBUF:5 * NBUF]

    wid = lax.axis_index("s") * NC + lax.axis_index("c")
    base = wid * PER_W

    def idx_copy(g, b):
        return pltpu.make_async_copy(
            idx_hbm.at[pl.ds(base + g * CHUNK, CHUNK)], idx_v[b], sems_i[b])

    def gather_copy(b):
        return pltpu.make_async_copy(
            table_hbm.at[idx_v[b]], rows_v[b], sems_g[b])

    def out_copy(g, b):
        return pltpu.make_async_copy(
            rows_v[b], out_hbm.at[pl.ds(base + g * CHUNK, CHUNK)], sems_o[b])

    for b in range(NBUF):
        idx_copy(b, b).start()
    for b in range(2):
        idx_copy(b, b).wait()
        gather_copy(b).start()

    def outer(go, carry):
        for b in range(NBUF):
            g = go * NBUF + b
            b2 = (b + 2) % NBUF

            gather_copy(b).wait()
            out_copy(g, b).start()
            @pl.when(go < NOUTER - 1)
            def _():
                idx_copy(g + NBUF, b).start()
            if b < 2:
                idx_copy(g + 2, b2).wait()
                @pl.when(go > 0)
                def _():
                    out_copy(g + 2 - NBUF, b2).wait()
                gather_copy(b2).start()
            else:
                @pl.when(go < NOUTER - 1)
                def _():
                    idx_copy(g + 2, b2).wait()
                    out_copy(g + 2 - NBUF, b2).wait()
                    gather_copy(b2).start()
        return carry

    lax.fori_loop(0, NOUTER, outer, 0)

    for g in range(NCHUNK - NBUF, NCHUNK):
        out_copy(g, g % NBUF).wait()


def kernel(x, table):
    flat = x.reshape(N).astype(jnp.int32)
    out = _gather_kernel(flat, table)
    return out.reshape(B, L, DIM)
